# Initial kernel scaffold; baseline (speedup 1.0000x reference)
import jax
import jax.numpy as jnp
from jax.experimental import pallas as pl


def kernel(im_features, ctx, token_prefix, token_suffix, W1, b1, W2, b2):
    raise NotImplementedError("write your pallas kernel here")


# speedup vs baseline: 1.0158x; 1.0158x over previous
import jax
import jax.numpy as jnp
from jax.experimental import pallas as pl
from jax.experimental.pallas import tpu as pltpu

_B = 8
_NC = 100
_NCTX = 10
_D = 512
_SUF = 66
_TKN = 77
_CT = 20


def _body(im_ref, ctx_ref, pre_ref, suf_ref, w1_ref, b1_ref, w2_ref, b2_ref,
          out_ref, ctxs_ref):
    c = pl.program_id(0)
    b = pl.program_id(1)

    @pl.when((c == 0) & (b == 0))
    def _():
        h = jnp.maximum(
            jnp.dot(im_ref[:], w1_ref[:], preferred_element_type=jnp.float32)
            + b1_ref[:], 0.0)
        bias = jnp.dot(h, w2_ref[:], preferred_element_type=jnp.float32) + b2_ref[:]
        ctxs_ref[:] = ctx_ref[:][None, :, :] + bias[:, None, :]

    out_ref[0, :, 0:1, :] = pre_ref[:]
    ctxb = ctxs_ref[b]
    out_ref[0, :, 1:1 + _NCTX, :] = jnp.broadcast_to(ctxb[None], (_CT, _NCTX, _D))
    out_ref[0, :, 1 + _NCTX:, :] = suf_ref[:]


def kernel(im_features, ctx, token_prefix, token_suffix, W1, b1, W2, b2):
    b1r = b1.reshape(1, -1)
    b2r = b2.reshape(1, -1)
    grid = (_NC // _CT, _B)
    return pl.pallas_call(
        _body,
        grid=grid,
        in_specs=[
            pl.BlockSpec((_B, _D), lambda c, b: (0, 0)),
            pl.BlockSpec((_NCTX, _D), lambda c, b: (0, 0)),
            pl.BlockSpec((_CT, 1, _D), lambda c, b: (c, 0, 0)),
            pl.BlockSpec((_CT, _SUF, _D), lambda c, b: (c, 0, 0)),
            pl.BlockSpec((_D, _D // 4), lambda c, b: (0, 0)),
            pl.BlockSpec((1, _D // 4), lambda c, b: (0, 0)),
            pl.BlockSpec((_D // 4, _D), lambda c, b: (0, 0)),
            pl.BlockSpec((1, _D), lambda c, b: (0, 0)),
        ],
        out_specs=pl.BlockSpec((1, _CT, _TKN, _D), lambda c, b: (b, c, 0, 0)),
        out_shape=jax.ShapeDtypeStruct((_B, _NC, _TKN, _D), jnp.float32),
        scratch_shapes=[pltpu.VMEM((_B, _NCTX, _D), jnp.float32)],
    )(im_features, ctx, token_prefix, token_suffix, W1, b1r, W2, b2r)
